# baseline (speedup 1.0000x reference)
import functools

import jax
import jax.numpy as jnp
from jax import lax
from jax.experimental import pallas as pl
from jax.experimental.pallas import tpu as pltpu
from jax.experimental.pallas import tpu_sc as plsc

N_NODES = 10000
D = 128
N_EDGES = 320000
N_LABEL = 100000

NC = 2
NS = 16
NW = NC * NS
L = 16

C = 128

N_PAD = 10112
TRASH_ROW = 10048
ROWS_PER_TILE = N_PAD // NS

CH_E = 80
E_PAD = NW * CH_E * C
CH_L = 26
NL_PAD = NW * CH_L * C
NP_ROWS = NL_PAD // 8

_MESH = plsc.VectorSubcoreMesh(
    core_axis_name="c", subcore_axis_name="s", num_cores=NC, num_subcores=NS
)

_CHUNKS_632 = ((0, 128), (128, 128), (256, 128), (384, 128), (512, 120))


def _fill_rows(buf, val16):
  @pl.loop(0, C)
  def _row(r):
    for j in range(D // L):
      buf[r, pl.ds(j * L, L)] = val16


def _zero_shared_slice(rbase, zsrc_v, dst_sh):
  for off, sz in _CHUNKS_632:
    pltpu.sync_copy(zsrc_v.at[pl.ds(0, sz)], dst_sh.at[pl.ds(rbase + off, sz)])


def _copy_out_slice(c, rbase, src_sh, bounce_v, out_hbm):
  for off, sz in _CHUNKS_632:
    pltpu.sync_copy(src_sh.at[pl.ds(rbase + off, sz)],
                    bounce_v.at[pl.ds(0, sz)])
    pltpu.sync_copy(bounce_v.at[pl.ds(0, sz)],
                    out_hbm.at[c, pl.ds(rbase + off, sz)])


def _sc_agg_body(x_hbm, src_hbm, dst_hbm, agg_out,
                 src_all, dst_all, rows_a, rows_b, acc_sh, sem_a, sem_b):
  c = lax.axis_index("c")
  s = lax.axis_index("s")
  wid = c * NS + s
  rbase = s * ROWS_PER_TILE

  pltpu.sync_copy(dst_hbm.at[pl.ds(wid * CH_E, CH_E)], dst_all)

  _fill_rows(rows_a, jnp.zeros((L,), jnp.float32))
  _zero_shared_slice(rbase, rows_a, acc_sh)
  plsc.subcore_barrier()

  half_ch = CH_E // 2
  for half in range(2):
    hbase = half * half_ch
    pltpu.sync_copy(src_hbm.at[pl.ds(wid * CH_E + hbase, half_ch)], src_all)

    pltpu.async_copy(x_hbm.at[src_all.at[0]], rows_a, sem_a)

    @pl.loop(0, half_ch // 2)
    def _pair(i):
      la = 2 * i
      pltpu.async_copy(x_hbm.at[src_all.at[la + 1]], rows_b, sem_b)
      pltpu.make_async_copy(x_hbm, rows_a, sem_a).wait()
      pltpu.sync_copy(rows_a, acc_sh.at[dst_all.at[hbase + la]], add=True)
      lnxt = jnp.minimum(la + 2, half_ch - 1)
      pltpu.async_copy(x_hbm.at[src_all.at[lnxt]], rows_a, sem_a)
      pltpu.make_async_copy(x_hbm, rows_b, sem_b).wait()
      pltpu.sync_copy(rows_b, acc_sh.at[dst_all.at[hbase + la + 1]], add=True)

    pltpu.make_async_copy(x_hbm, rows_a, sem_a).wait()

  plsc.subcore_barrier()
  _copy_out_slice(c, rbase, acc_sh, rows_a, agg_out)


_sc_agg = pl.kernel(
    _sc_agg_body,
    out_type=jax.ShapeDtypeStruct((NC, N_PAD, D), jnp.float32),
    mesh=_MESH,
    scratch_types=[
        pltpu.VMEM((CH_E // 2, C), jnp.int32),
        pltpu.VMEM((CH_E, C), jnp.int32),
        pltpu.VMEM((C, D), jnp.float32),
        pltpu.VMEM((C, D), jnp.float32),
        pltpu.VMEM_SHARED((N_PAD, D), jnp.float32),
        pltpu.SemaphoreType.DMA,
        pltpu.SemaphoreType.DMA,
    ],
)


def _sc_cnt_body(dst_hbm, cnt_out, dst_all, ones_v, cnt_sh, sem):
  c = lax.axis_index("c")
  s = lax.axis_index("s")
  wid = c * NS + s
  rbase = s * ROWS_PER_TILE

  pltpu.sync_copy(dst_hbm.at[pl.ds(wid * CH_E, CH_E)], dst_all)

  _fill_rows(ones_v, jnp.zeros((L,), jnp.float32))
  _zero_shared_slice(rbase, ones_v, cnt_sh)
  _fill_rows(ones_v, jnp.ones((L,), jnp.float32))
  plsc.subcore_barrier()

  @pl.loop(0, CH_E)
  def _edge_chunk(i):
    pltpu.sync_copy(ones_v, cnt_sh.at[dst_all.at[i]], add=True)

  plsc.subcore_barrier()
  _copy_out_slice(c, rbase, cnt_sh, ones_v, cnt_out)


_sc_cnt = pl.kernel(
    _sc_cnt_body,
    out_type=jax.ShapeDtypeStruct((NC, N_PAD, D), jnp.float32),
    mesh=_MESH,
    scratch_types=[
        pltpu.VMEM((CH_E, C), jnp.int32),
        pltpu.VMEM((C, D), jnp.float32),
        pltpu.VMEM_SHARED((N_PAD, D), jnp.float32),
        pltpu.SemaphoreType.DMA,
    ],
)


def _score_chunk(rows_a, rows_b, pack_v):
  @pl.loop(0, C // L)
  def _group(g):
    for r16 in range(L):
      r = g * L + r16
      acc = rows_a[r, pl.ds(0, L)] * rows_b[r, pl.ds(0, L)]
      for j in range(1, D // L):
        acc += rows_a[r, pl.ds(j * L, L)] * rows_b[r, pl.ds(j * L, L)]
      pack_v[2 * g + r16 // 8, pl.ds((r16 % 8) * L, L)] = acc


def _sc_score_body(h_hbm, ia_hbm, ib_hbm, part_out,
                   ia_all, ib_all, rows_a0, rows_b0, rows_a1, rows_b1,
                   pack_v, sem0, sem1):
  c = lax.axis_index("c")
  s = lax.axis_index("s")
  wid = c * NS + s

  pltpu.sync_copy(ia_hbm.at[wid], ia_all)
  pltpu.sync_copy(ib_hbm.at[wid], ib_all)

  pltpu.async_copy(h_hbm.at[ia_all.at[0]], rows_a0, sem0)
  pltpu.async_copy(h_hbm.at[ib_all.at[0]], rows_b0, sem0)

  pbase0 = wid * (CH_L * C // 8)

  @pl.loop(0, CH_L // 2)
  def _pair(i):
    ca = 2 * i
    pltpu.async_copy(h_hbm.at[ia_all.at[ca + 1]], rows_a1, sem1)
    pltpu.async_copy(h_hbm.at[ib_all.at[ca + 1]], rows_b1, sem1)
    pltpu.make_async_copy(h_hbm, rows_a0, sem0).wait()
    pltpu.make_async_copy(h_hbm, rows_b0, sem0).wait()
    _score_chunk(rows_a0, rows_b0, pack_v)
    pltpu.sync_copy(
        pack_v, part_out.at[pl.ds(pbase0 + i * (2 * (C // 8)), C // 8)])
    cnxt = jnp.minimum(ca + 2, CH_L - 1)
    pltpu.async_copy(h_hbm.at[ia_all.at[cnxt]], rows_a0, sem0)
    pltpu.async_copy(h_hbm.at[ib_all.at[cnxt]], rows_b0, sem0)
    pltpu.make_async_copy(h_hbm, rows_a1, sem1).wait()
    pltpu.make_async_copy(h_hbm, rows_b1, sem1).wait()
    _score_chunk(rows_a1, rows_b1, pack_v)
    pltpu.sync_copy(
        pack_v,
        part_out.at[pl.ds(pbase0 + i * (2 * (C // 8)) + C // 8, C // 8)])

  pltpu.make_async_copy(h_hbm, rows_a0, sem0).wait()
  pltpu.make_async_copy(h_hbm, rows_b0, sem0).wait()


_sc_score = pl.kernel(
    _sc_score_body,
    out_type=jax.ShapeDtypeStruct((NP_ROWS, D), jnp.float32),
    mesh=_MESH,
    scratch_types=[
        pltpu.VMEM((CH_L, C), jnp.int32),
        pltpu.VMEM((CH_L, C), jnp.int32),
        pltpu.VMEM((C, D), jnp.float32),
        pltpu.VMEM((C, D), jnp.float32),
        pltpu.VMEM((C, D), jnp.float32),
        pltpu.VMEM((C, D), jnp.float32),
        pltpu.VMEM((C // 8, D), jnp.float32),
        pltpu.SemaphoreType.DMA,
        pltpu.SemaphoreType.DMA,
    ],
)


def _tc_dense_body(relu, agg_ref, cnt_ref, x_ref, wl_ref, wr_ref, b_ref, out_ref):
  agg = agg_ref[0] + agg_ref[1]
  cnt = cnt_ref[0, :, 0:1] + cnt_ref[1, :, 0:1]
  mean = agg / jnp.maximum(cnt, 1.0)
  h = (jnp.dot(mean, wl_ref[:], preferred_element_type=jnp.float32)
       + jnp.dot(x_ref[:], wr_ref[:], preferred_element_type=jnp.float32)
       + b_ref[:])
  if relu:
    h = jnp.maximum(h, 0.0)
  out_ref[:] = h


def _tc_dense(agg, cnt, x, w_l, w_r, b, relu):
  grid = 4
  r = N_PAD // grid
  return pl.pallas_call(
      functools.partial(_tc_dense_body, relu),
      grid=(grid,),
      in_specs=[
          pl.BlockSpec((NC, r, D), lambda i: (0, i, 0)),
          pl.BlockSpec((NC, r, D), lambda i: (0, i, 0)),
          pl.BlockSpec((r, D), lambda i: (i, 0)),
          pl.BlockSpec((D, D), lambda i: (0, 0)),
          pl.BlockSpec((D, D), lambda i: (0, 0)),
          pl.BlockSpec((1, D), lambda i: (0, 0)),
      ],
      out_specs=pl.BlockSpec((r, D), lambda i: (i, 0)),
      out_shape=jax.ShapeDtypeStruct((N_PAD, D), jnp.float32),
  )(agg, cnt, x, w_l, w_r, b)


def _tc_reduce_body(part_ref, out_ref):
  cols = [jnp.sum(part_ref[:, q * L:(q + 1) * L], axis=1, keepdims=True)
          for q in range(8)]
  out_ref[:] = jnp.concatenate(cols, axis=1)


def _tc_reduce(part):
  grid = 8
  r = NP_ROWS // grid
  return pl.pallas_call(
      _tc_reduce_body,
      grid=(grid,),
      in_specs=[pl.BlockSpec((r, D), lambda i: (i, 0))],
      out_specs=pl.BlockSpec((r, 8), lambda i: (i, 0)),
      out_shape=jax.ShapeDtypeStruct((NP_ROWS, 8), jnp.float32),
  )(part)


def kernel(node_feature, edge_index, edge_label_index, W1_l, W1_r, b1, W2_l, W2_r, b2):
  ei = edge_index.astype(jnp.int32)
  src = jnp.concatenate(
      [ei[0], jnp.zeros((E_PAD - N_EDGES,), jnp.int32)]).reshape(NW * CH_E, C)
  dst = jnp.concatenate(
      [ei[1], jnp.full((E_PAD - N_EDGES,), TRASH_ROW, jnp.int32)]
  ).reshape(NW * CH_E, C)
  eli = edge_label_index.astype(jnp.int32)
  ia = jnp.concatenate(
      [eli[0], jnp.zeros((NL_PAD - N_LABEL,), jnp.int32)]).reshape(NW, CH_L, C)
  ib = jnp.concatenate(
      [eli[1], jnp.zeros((NL_PAD - N_LABEL,), jnp.int32)]).reshape(NW, CH_L, C)

  x = jnp.concatenate(
      [node_feature, jnp.zeros((N_PAD - N_NODES, D), jnp.float32)])

  cnt = _sc_cnt(dst)
  agg1 = _sc_agg(x, src, dst)
  h1 = _tc_dense(agg1, cnt, x, W1_l, W1_r, b1.reshape(1, D), relu=True)
  agg2 = _sc_agg(h1, src, dst)
  h2 = _tc_dense(agg2, cnt, h1, W2_l, W2_r, b2.reshape(1, D), relu=False)
  part = _sc_score(h2, ia, ib)
  pred = _tc_reduce(part)
  return pred.reshape(NL_PAD)[:N_LABEL]


# speedup vs baseline: 1.0881x; 1.0881x over previous
import functools

import jax
import jax.numpy as jnp
from jax import lax
from jax.experimental import pallas as pl
from jax.experimental.pallas import tpu as pltpu
from jax.experimental.pallas import tpu_sc as plsc

N_NODES = 10000
D = 128
N_EDGES = 320000
N_LABEL = 100000

NC = 2
NS = 16
NW = NC * NS
L = 16

C = 128

N_PAD = 10112
TRASH_ROW = 10048
ROWS_PER_TILE = N_PAD // NS

CH_E = 80
E_PAD = NW * CH_E * C
CH_L = 26
NL_PAD = NW * CH_L * C
NP_ROWS = NL_PAD // 8

_MESH = plsc.VectorSubcoreMesh(
    core_axis_name="c", subcore_axis_name="s", num_cores=NC, num_subcores=NS
)

_CHUNKS_632 = ((0, 128), (128, 128), (256, 128), (384, 128), (512, 120))


def _fill_rows(buf, val16):
  @pl.loop(0, C)
  def _row(r):
    for j in range(D // L):
      buf[r, pl.ds(j * L, L)] = val16


def _zero_shared_slice(rbase, zsrc_v, dst_sh):
  for off, sz in _CHUNKS_632:
    pltpu.sync_copy(zsrc_v.at[pl.ds(0, sz)], dst_sh.at[pl.ds(rbase + off, sz)])


def _copy_out_slice(c, rbase, src_sh, bounce_v, out_hbm):
  for off, sz in _CHUNKS_632:
    pltpu.sync_copy(src_sh.at[pl.ds(rbase + off, sz)],
                    bounce_v.at[pl.ds(0, sz)])
    pltpu.sync_copy(bounce_v.at[pl.ds(0, sz)],
                    out_hbm.at[c, pl.ds(rbase + off, sz)])


def _sc_agg_body(x_hbm, src_hbm, dst_hbm, agg_out,
                 src_all, dst_all, rows_v, acc_sh, sem):
  c = lax.axis_index("c")
  s = lax.axis_index("s")
  wid = c * NS + s
  rbase = s * ROWS_PER_TILE

  pltpu.sync_copy(src_hbm.at[pl.ds(wid * (CH_E * C), CH_E * C)], src_all)
  pltpu.sync_copy(dst_hbm.at[pl.ds(wid * CH_E, CH_E)], dst_all)

  _fill_rows(rows_v, jnp.zeros((L,), jnp.float32))
  _zero_shared_slice(rbase, rows_v, acc_sh)
  plsc.subcore_barrier()

  @pl.loop(0, CH_E)
  def _edge_chunk(i):
    pltpu.async_copy(
        x_hbm.at[src_all.at[pl.ds(i * C, C)]], rows_v, sem).wait()
    pltpu.sync_copy(rows_v, acc_sh.at[dst_all.at[i]], add=True)

  plsc.subcore_barrier()
  _copy_out_slice(c, rbase, acc_sh, rows_v, agg_out)


_sc_agg = pl.kernel(
    _sc_agg_body,
    out_type=jax.ShapeDtypeStruct((NC, N_PAD, D), jnp.float32),
    mesh=_MESH,
    scratch_types=[
        pltpu.VMEM((CH_E * C,), jnp.int32),
        pltpu.VMEM((CH_E, C), jnp.int32),
        pltpu.VMEM((C, D), jnp.float32),
        pltpu.VMEM_SHARED((N_PAD, D), jnp.float32),
        pltpu.SemaphoreType.DMA,
    ],
)


def _sc_cnt_body(dst_hbm, cnt_out, dst_all, ones_v, cnt_sh, sem):
  c = lax.axis_index("c")
  s = lax.axis_index("s")
  wid = c * NS + s
  rbase = s * ROWS_PER_TILE

  pltpu.sync_copy(dst_hbm.at[pl.ds(wid * CH_E, CH_E)], dst_all)

  _fill_rows(ones_v, jnp.zeros((L,), jnp.float32))
  _zero_shared_slice(rbase, ones_v, cnt_sh)
  _fill_rows(ones_v, jnp.ones((L,), jnp.float32))
  plsc.subcore_barrier()

  @pl.loop(0, CH_E)
  def _edge_chunk(i):
    pltpu.sync_copy(ones_v, cnt_sh.at[dst_all.at[i]], add=True)

  plsc.subcore_barrier()
  _copy_out_slice(c, rbase, cnt_sh, ones_v, cnt_out)


_sc_cnt = pl.kernel(
    _sc_cnt_body,
    out_type=jax.ShapeDtypeStruct((NC, N_PAD, D), jnp.float32),
    mesh=_MESH,
    scratch_types=[
        pltpu.VMEM((CH_E, C), jnp.int32),
        pltpu.VMEM((C, D), jnp.float32),
        pltpu.VMEM_SHARED((N_PAD, D), jnp.float32),
        pltpu.SemaphoreType.DMA,
    ],
)


def _score_chunk(rows_a, rows_b, pack_v):
  @pl.loop(0, C // L)
  def _group(g):
    for r16 in range(L):
      r = g * L + r16
      acc = rows_a[r, pl.ds(0, L)] * rows_b[r, pl.ds(0, L)]
      for j in range(1, D // L):
        acc += rows_a[r, pl.ds(j * L, L)] * rows_b[r, pl.ds(j * L, L)]
      pack_v[2 * g + r16 // 8, pl.ds((r16 % 8) * L, L)] = acc


def _sc_score_body(h_hbm, ia_hbm, ib_hbm, part_out,
                   ia_all, ib_all, rows_a, rows_b, pack_v, sem):
  c = lax.axis_index("c")
  s = lax.axis_index("s")
  wid = c * NS + s

  pltpu.sync_copy(ia_hbm.at[wid], ia_all)
  pltpu.sync_copy(ib_hbm.at[wid], ib_all)

  pbase0 = wid * (CH_L * C // 8)

  @pl.loop(0, CH_L)
  def _label_chunk(i):
    da = pltpu.async_copy(h_hbm.at[ia_all.at[i]], rows_a, sem)
    db = pltpu.async_copy(h_hbm.at[ib_all.at[i]], rows_b, sem)
    da.wait()
    db.wait()
    _score_chunk(rows_a, rows_b, pack_v)
    pltpu.sync_copy(
        pack_v, part_out.at[pl.ds(pbase0 + i * (C // 8), C // 8)])


_sc_score = pl.kernel(
    _sc_score_body,
    out_type=jax.ShapeDtypeStruct((NP_ROWS, D), jnp.float32),
    mesh=_MESH,
    scratch_types=[
        pltpu.VMEM((CH_L, C), jnp.int32),
        pltpu.VMEM((CH_L, C), jnp.int32),
        pltpu.VMEM((C, D), jnp.float32),
        pltpu.VMEM((C, D), jnp.float32),
        pltpu.VMEM((C // 8, D), jnp.float32),
        pltpu.SemaphoreType.DMA,
    ],
)


def _tc_dense_body(relu, agg_ref, cnt_ref, x_ref, wl_ref, wr_ref, b_ref, out_ref):
  agg = agg_ref[0] + agg_ref[1]
  cnt = cnt_ref[0, :, 0:1] + cnt_ref[1, :, 0:1]
  mean = agg / jnp.maximum(cnt, 1.0)
  h = (jnp.dot(mean, wl_ref[:], preferred_element_type=jnp.float32)
       + jnp.dot(x_ref[:], wr_ref[:], preferred_element_type=jnp.float32)
       + b_ref[:])
  if relu:
    h = jnp.maximum(h, 0.0)
  out_ref[:] = h


def _tc_dense(agg, cnt, x, w_l, w_r, b, relu):
  grid = 4
  r = N_PAD // grid
  return pl.pallas_call(
      functools.partial(_tc_dense_body, relu),
      grid=(grid,),
      in_specs=[
          pl.BlockSpec((NC, r, D), lambda i: (0, i, 0)),
          pl.BlockSpec((NC, r, D), lambda i: (0, i, 0)),
          pl.BlockSpec((r, D), lambda i: (i, 0)),
          pl.BlockSpec((D, D), lambda i: (0, 0)),
          pl.BlockSpec((D, D), lambda i: (0, 0)),
          pl.BlockSpec((1, D), lambda i: (0, 0)),
      ],
      out_specs=pl.BlockSpec((r, D), lambda i: (i, 0)),
      out_shape=jax.ShapeDtypeStruct((N_PAD, D), jnp.float32),
  )(agg, cnt, x, w_l, w_r, b)


def _tc_reduce_body(part_ref, out_ref):
  cols = [jnp.sum(part_ref[:, q * L:(q + 1) * L], axis=1, keepdims=True)
          for q in range(8)]
  out_ref[:] = jnp.concatenate(cols, axis=1)


def _tc_reduce(part):
  grid = 8
  r = NP_ROWS // grid
  return pl.pallas_call(
      _tc_reduce_body,
      grid=(grid,),
      in_specs=[pl.BlockSpec((r, D), lambda i: (i, 0))],
      out_specs=pl.BlockSpec((r, 8), lambda i: (i, 0)),
      out_shape=jax.ShapeDtypeStruct((NP_ROWS, 8), jnp.float32),
  )(part)


def kernel(node_feature, edge_index, edge_label_index, W1_l, W1_r, b1, W2_l, W2_r, b2):
  ei = edge_index.astype(jnp.int32)
  src = jnp.concatenate(
      [ei[0], jnp.zeros((E_PAD - N_EDGES,), jnp.int32)])
  dst = jnp.concatenate(
      [ei[1], jnp.full((E_PAD - N_EDGES,), TRASH_ROW, jnp.int32)]
  ).reshape(NW * CH_E, C)
  eli = edge_label_index.astype(jnp.int32)
  ia = jnp.concatenate(
      [eli[0], jnp.zeros((NL_PAD - N_LABEL,), jnp.int32)]).reshape(NW, CH_L, C)
  ib = jnp.concatenate(
      [eli[1], jnp.zeros((NL_PAD - N_LABEL,), jnp.int32)]).reshape(NW, CH_L, C)

  x = jnp.concatenate(
      [node_feature, jnp.zeros((N_PAD - N_NODES, D), jnp.float32)])

  cnt = _sc_cnt(dst)
  agg1 = _sc_agg(x, src, dst)
  h1 = _tc_dense(agg1, cnt, x, W1_l, W1_r, b1.reshape(1, D), relu=True)
  agg2 = _sc_agg(h1, src, dst)
  h2 = _tc_dense(agg2, cnt, h1, W2_l, W2_r, b2.reshape(1, D), relu=False)
  part = _sc_score(h2, ia, ib)
  pred = _tc_reduce(part)
  return pred.reshape(NL_PAD)[:N_LABEL]


# speedup vs baseline: 1.6171x; 1.4862x over previous
import functools

import jax
import jax.numpy as jnp
from jax import lax
from jax.experimental import pallas as pl
from jax.experimental.pallas import tpu as pltpu
from jax.experimental.pallas import tpu_sc as plsc

N_NODES = 10000
D = 128
N_EDGES = 320000
N_LABEL = 100000

NC = 2
NS = 16
NW = NC * NS
L = 16

C = 128

N_PAD = 10112
TRASH_ROW = 10048
ROWS_PER_TILE = N_PAD // NS

CH_E = 79
E_PAD = NW * CH_E * C
CH_L = 25
NL_PAD = NW * CH_L * C
NP_ROWS = NL_PAD // 8

_MESH = plsc.VectorSubcoreMesh(
    core_axis_name="c", subcore_axis_name="s", num_cores=NC, num_subcores=NS
)

_CHUNKS_632 = ((0, 128), (128, 128), (256, 128), (384, 128), (512, 120))


def _fill_rows(buf, val16):
  @pl.loop(0, C)
  def _row(r):
    for j in range(D // L):
      buf[r, pl.ds(j * L, L)] = val16


def _zero_shared_slice(rbase, zsrc_v, dst_sh):
  for off, sz in _CHUNKS_632:
    pltpu.sync_copy(zsrc_v.at[pl.ds(0, sz)], dst_sh.at[pl.ds(rbase + off, sz)])


def _copy_out_slice(c, rbase, src_sh, bounce_v, out_hbm):
  for off, sz in _CHUNKS_632:
    pltpu.sync_copy(src_sh.at[pl.ds(rbase + off, sz)],
                    bounce_v.at[pl.ds(0, sz)])
    pltpu.sync_copy(bounce_v.at[pl.ds(0, sz)],
                    out_hbm.at[c, pl.ds(rbase + off, sz)])


def _sc_agg_body(x_hbm, src_hbm, dst_hbm, agg_out,
                 src_v, dst_all, rows_v, acc_sh, sem):
  c = lax.axis_index("c")
  s = lax.axis_index("s")
  wid = c * NS + s
  rbase = s * ROWS_PER_TILE

  pltpu.sync_copy(dst_hbm.at[wid], dst_all)

  _fill_rows(rows_v, jnp.zeros((L,), jnp.float32))
  _zero_shared_slice(rbase, rows_v, acc_sh)
  plsc.subcore_barrier()

  ebase0 = wid * (CH_E * C)

  @pl.loop(0, CH_E)
  def _edge_chunk(i):
    pltpu.sync_copy(src_hbm.at[pl.ds(ebase0 + i * C, C)], src_v)
    pltpu.async_copy(x_hbm.at[src_v], rows_v, sem).wait()
    pltpu.sync_copy(rows_v, acc_sh.at[dst_all.at[i]], add=True)

  plsc.subcore_barrier()
  _copy_out_slice(c, rbase, acc_sh, rows_v, agg_out)


_sc_agg = pl.kernel(
    _sc_agg_body,
    out_type=jax.ShapeDtypeStruct((NC, N_PAD, D), jnp.float32),
    mesh=_MESH,
    scratch_types=[
        pltpu.VMEM((C,), jnp.int32),
        pltpu.VMEM((CH_E, C), jnp.int32),
        pltpu.VMEM((C, D), jnp.float32),
        pltpu.VMEM_SHARED((N_PAD, D), jnp.float32),
        pltpu.SemaphoreType.DMA,
    ],
)


def _sc_cnt_body(dst_hbm, cnt_out, dst_all, ones_v, cnt_sh, sem):
  c = lax.axis_index("c")
  s = lax.axis_index("s")
  wid = c * NS + s
  rbase = s * ROWS_PER_TILE

  pltpu.sync_copy(dst_hbm.at[wid], dst_all)

  _fill_rows(ones_v, jnp.zeros((L,), jnp.float32))
  _zero_shared_slice(rbase, ones_v, cnt_sh)
  _fill_rows(ones_v, jnp.ones((L,), jnp.float32))
  plsc.subcore_barrier()

  @pl.loop(0, CH_E)
  def _edge_chunk(i):
    pltpu.sync_copy(ones_v, cnt_sh.at[dst_all.at[i]], add=True)

  plsc.subcore_barrier()
  _copy_out_slice(c, rbase, cnt_sh, ones_v, cnt_out)


_sc_cnt = pl.kernel(
    _sc_cnt_body,
    out_type=jax.ShapeDtypeStruct((NC, N_PAD, D), jnp.float32),
    mesh=_MESH,
    scratch_types=[
        pltpu.VMEM((CH_E, C), jnp.int32),
        pltpu.VMEM((C, D), jnp.float32),
        pltpu.VMEM_SHARED((N_PAD, D), jnp.float32),
        pltpu.SemaphoreType.DMA,
    ],
)


def _sc_score_body(h_hbm, ia_hbm, ib_hbm, part_out,
                   ia_v, ib_v, rows_a, rows_b, pack_v, sem):
  c = lax.axis_index("c")
  s = lax.axis_index("s")
  wid = c * NS + s
  base0 = wid * (CH_L * C)

  @pl.loop(0, CH_L)
  def _label_chunk(i):
    base = base0 + i * C
    pltpu.sync_copy(ia_hbm.at[pl.ds(base, C)], ia_v)
    pltpu.sync_copy(ib_hbm.at[pl.ds(base, C)], ib_v)
    da = pltpu.async_copy(h_hbm.at[ia_v], rows_a, sem)
    db = pltpu.async_copy(h_hbm.at[ib_v], rows_b, sem)
    da.wait()
    db.wait()

    @pl.loop(0, C // L)
    def _group(g):
      for r16 in range(L):
        r = g * L + r16
        acc = rows_a[r, pl.ds(0, L)] * rows_b[r, pl.ds(0, L)]
        for j in range(1, D // L):
          acc += rows_a[r, pl.ds(j * L, L)] * rows_b[r, pl.ds(j * L, L)]
        pack_v[2 * g + r16 // 8, pl.ds((r16 % 8) * L, L)] = acc

    pbase = wid * (CH_L * C // 8) + i * (C // 8)
    pltpu.sync_copy(pack_v, part_out.at[pl.ds(pbase, C // 8)])


_sc_score = pl.kernel(
    _sc_score_body,
    out_type=jax.ShapeDtypeStruct((NP_ROWS, D), jnp.float32),
    mesh=_MESH,
    scratch_types=[
        pltpu.VMEM((C,), jnp.int32),
        pltpu.VMEM((C,), jnp.int32),
        pltpu.VMEM((C, D), jnp.float32),
        pltpu.VMEM((C, D), jnp.float32),
        pltpu.VMEM((C // 8, D), jnp.float32),
        pltpu.SemaphoreType.DMA,
    ],
)


def _tc_dense_body(relu, agg_ref, cnt_ref, x_ref, wl_ref, wr_ref, b_ref, out_ref):
  agg = agg_ref[0] + agg_ref[1]
  cnt = cnt_ref[0, :, 0:1] + cnt_ref[1, :, 0:1]
  mean = agg / jnp.maximum(cnt, 1.0)
  h = (jnp.dot(mean, wl_ref[:], preferred_element_type=jnp.float32)
       + jnp.dot(x_ref[:], wr_ref[:], preferred_element_type=jnp.float32)
       + b_ref[:])
  if relu:
    h = jnp.maximum(h, 0.0)
  out_ref[:] = h


def _tc_dense(agg, cnt, x, w_l, w_r, b, relu):
  grid = 4
  r = N_PAD // grid
  return pl.pallas_call(
      functools.partial(_tc_dense_body, relu),
      grid=(grid,),
      in_specs=[
          pl.BlockSpec((NC, r, D), lambda i: (0, i, 0)),
          pl.BlockSpec((NC, r, D), lambda i: (0, i, 0)),
          pl.BlockSpec((r, D), lambda i: (i, 0)),
          pl.BlockSpec((D, D), lambda i: (0, 0)),
          pl.BlockSpec((D, D), lambda i: (0, 0)),
          pl.BlockSpec((1, D), lambda i: (0, 0)),
      ],
      out_specs=pl.BlockSpec((r, D), lambda i: (i, 0)),
      out_shape=jax.ShapeDtypeStruct((N_PAD, D), jnp.float32),
  )(agg, cnt, x, w_l, w_r, b)


def _tc_reduce_body(part_ref, out_ref):
  cols = [jnp.sum(part_ref[:, q * L:(q + 1) * L], axis=1, keepdims=True)
          for q in range(8)]
  out_ref[:] = jnp.concatenate(cols, axis=1)


def _tc_reduce(part):
  grid = 8
  r = NP_ROWS // grid
  return pl.pallas_call(
      _tc_reduce_body,
      grid=(grid,),
      in_specs=[pl.BlockSpec((r, D), lambda i: (i, 0))],
      out_specs=pl.BlockSpec((r, 8), lambda i: (i, 0)),
      out_shape=jax.ShapeDtypeStruct((NP_ROWS, 8), jnp.float32),
  )(part)


def kernel(node_feature, edge_index, edge_label_index, W1_l, W1_r, b1, W2_l, W2_r, b2):
  ei = edge_index.astype(jnp.int32)
  src = jnp.concatenate(
      [ei[0], jnp.zeros((E_PAD - N_EDGES,), jnp.int32)])
  dst = jnp.concatenate(
      [ei[1], jnp.full((E_PAD - N_EDGES,), TRASH_ROW, jnp.int32)]
  ).reshape(NW, CH_E, C)
  eli = edge_label_index.astype(jnp.int32)
  ia = jnp.concatenate([eli[0], jnp.zeros((NL_PAD - N_LABEL,), jnp.int32)])
  ib = jnp.concatenate([eli[1], jnp.zeros((NL_PAD - N_LABEL,), jnp.int32)])

  x = jnp.concatenate(
      [node_feature, jnp.zeros((N_PAD - N_NODES, D), jnp.float32)])

  cnt = _sc_cnt(dst)
  agg1 = _sc_agg(x, src, dst)
  h1 = _tc_dense(agg1, cnt, x, W1_l, W1_r, b1.reshape(1, D), relu=True)
  agg2 = _sc_agg(h1, src, dst)
  h2 = _tc_dense(agg2, cnt, h1, W2_l, W2_r, b2.reshape(1, D), relu=False)
  part = _sc_score(h2, ia, ib)
  pred = _tc_reduce(part)
  return pred.reshape(NL_PAD)[:N_LABEL]


# speedup vs baseline: 1.7307x; 1.0702x over previous
import functools

import jax
import jax.numpy as jnp
from jax import lax
from jax.experimental import pallas as pl
from jax.experimental.pallas import tpu as pltpu
from jax.experimental.pallas import tpu_sc as plsc

N_NODES = 10000
D = 128
N_EDGES = 320000
N_LABEL = 100000

NC = 2
NS = 16
NW = NC * NS
L = 16

C = 128

N_PAD = 10112
TRASH_ROW = 10048
ROWS_PER_TILE = N_PAD // NS

CH_E = 79
E_PAD = NW * CH_E * C
CH_L = 25
NL_PAD = NW * CH_L * C
NP_ROWS = NL_PAD // 8

_MESH = plsc.VectorSubcoreMesh(
    core_axis_name="c", subcore_axis_name="s", num_cores=NC, num_subcores=NS
)

_CHUNKS_632 = ((0, 128), (128, 128), (256, 128), (384, 128), (512, 120))


def _fill_rows(buf, val16):
  @pl.loop(0, C)
  def _row(r):
    for j in range(D // L):
      buf[r, pl.ds(j * L, L)] = val16


def _zero_shared_slice(rbase, zsrc_v, dst_sh):
  for off, sz in _CHUNKS_632:
    pltpu.sync_copy(zsrc_v.at[pl.ds(0, sz)], dst_sh.at[pl.ds(rbase + off, sz)])


def _copy_out_slice(c, rbase, src_sh, bounce_v, out_hbm):
  for off, sz in _CHUNKS_632:
    pltpu.sync_copy(src_sh.at[pl.ds(rbase + off, sz)],
                    bounce_v.at[pl.ds(0, sz)])
    pltpu.sync_copy(bounce_v.at[pl.ds(0, sz)],
                    out_hbm.at[c, pl.ds(rbase + off, sz)])


def _sc_agg_body(x_hbm, src_hbm, dst_hbm, agg_out,
                 src_v, src_vb, dst_all, rows_v, acc_sh, sem, sem_ia, sem_ib):
  c = lax.axis_index("c")
  s = lax.axis_index("s")
  wid = c * NS + s
  rbase = s * ROWS_PER_TILE

  pltpu.sync_copy(dst_hbm.at[wid], dst_all)

  _fill_rows(rows_v, jnp.zeros((L,), jnp.float32))
  _zero_shared_slice(rbase, rows_v, acc_sh)
  plsc.subcore_barrier()

  ebase0 = wid * (CH_E * C)

  pltpu.async_copy(src_hbm.at[pl.ds(ebase0, C)], src_v, sem_ia)

  @pl.loop(0, CH_E // 2)
  def _edge_pair(i):
    pltpu.async_copy(
        src_hbm.at[pl.ds(ebase0 + (2 * i + 1) * C, C)], src_vb, sem_ib)
    pltpu.make_async_copy(src_hbm, src_v, sem_ia).wait()
    pltpu.async_copy(x_hbm.at[src_v], rows_v, sem).wait()
    pltpu.sync_copy(rows_v, acc_sh.at[dst_all.at[2 * i]], add=True)
    pltpu.async_copy(
        src_hbm.at[pl.ds(ebase0 + (2 * i + 2) * C, C)], src_v, sem_ia)
    pltpu.make_async_copy(src_hbm, src_vb, sem_ib).wait()
    pltpu.async_copy(x_hbm.at[src_vb], rows_v, sem).wait()
    pltpu.sync_copy(rows_v, acc_sh.at[dst_all.at[2 * i + 1]], add=True)

  pltpu.make_async_copy(src_hbm, src_v, sem_ia).wait()
  pltpu.async_copy(x_hbm.at[src_v], rows_v, sem).wait()
  pltpu.sync_copy(rows_v, acc_sh.at[dst_all.at[CH_E - 1]], add=True)

  plsc.subcore_barrier()
  _copy_out_slice(c, rbase, acc_sh, rows_v, agg_out)


_sc_agg = pl.kernel(
    _sc_agg_body,
    out_type=jax.ShapeDtypeStruct((NC, N_PAD, D), jnp.float32),
    mesh=_MESH,
    scratch_types=[
        pltpu.VMEM((C,), jnp.int32),
        pltpu.VMEM((C,), jnp.int32),
        pltpu.VMEM((CH_E, C), jnp.int32),
        pltpu.VMEM((C, D), jnp.float32),
        pltpu.VMEM_SHARED((N_PAD, D), jnp.float32),
        pltpu.SemaphoreType.DMA,
        pltpu.SemaphoreType.DMA,
        pltpu.SemaphoreType.DMA,
    ],
)


def _sc_cnt_body(dst_hbm, cnt_out, dst_all, ones_v, cnt_sh, sem):
  c = lax.axis_index("c")
  s = lax.axis_index("s")
  wid = c * NS + s
  rbase = s * ROWS_PER_TILE

  pltpu.sync_copy(dst_hbm.at[wid], dst_all)

  _fill_rows(ones_v, jnp.zeros((L,), jnp.float32))
  _zero_shared_slice(rbase, ones_v, cnt_sh)
  _fill_rows(ones_v, jnp.ones((L,), jnp.float32))
  plsc.subcore_barrier()

  @pl.loop(0, CH_E)
  def _edge_chunk(i):
    pltpu.sync_copy(ones_v, cnt_sh.at[dst_all.at[i]], add=True)

  plsc.subcore_barrier()
  _copy_out_slice(c, rbase, cnt_sh, ones_v, cnt_out)


_sc_cnt = pl.kernel(
    _sc_cnt_body,
    out_type=jax.ShapeDtypeStruct((NC, N_PAD, D), jnp.float32),
    mesh=_MESH,
    scratch_types=[
        pltpu.VMEM((CH_E, C), jnp.int32),
        pltpu.VMEM((C, D), jnp.float32),
        pltpu.VMEM_SHARED((N_PAD, D), jnp.float32),
        pltpu.SemaphoreType.DMA,
    ],
)


def _sc_score_body(h_hbm, ia_hbm, ib_hbm, part_out,
                   ia_all, ib_all, rows_a, rows_b, pack_v, sem):
  c = lax.axis_index("c")
  s = lax.axis_index("s")
  wid = c * NS + s

  pltpu.sync_copy(ia_hbm.at[wid], ia_all)
  pltpu.sync_copy(ib_hbm.at[wid], ib_all)

  @pl.loop(0, CH_L)
  def _label_chunk(i):
    da = pltpu.async_copy(h_hbm.at[ia_all.at[i]], rows_a, sem)
    db = pltpu.async_copy(h_hbm.at[ib_all.at[i]], rows_b, sem)
    da.wait()
    db.wait()

    @pl.loop(0, C // L)
    def _group(g):
      for r16 in range(L):
        r = g * L + r16
        acc = rows_a[r, pl.ds(0, L)] * rows_b[r, pl.ds(0, L)]
        for j in range(1, D // L):
          acc += rows_a[r, pl.ds(j * L, L)] * rows_b[r, pl.ds(j * L, L)]
        pack_v[2 * g + r16 // 8, pl.ds((r16 % 8) * L, L)] = acc

    pbase = wid * (CH_L * C // 8) + i * (C // 8)
    pltpu.sync_copy(pack_v, part_out.at[pl.ds(pbase, C // 8)])


_sc_score = pl.kernel(
    _sc_score_body,
    out_type=jax.ShapeDtypeStruct((NP_ROWS, D), jnp.float32),
    mesh=_MESH,
    scratch_types=[
        pltpu.VMEM((CH_L, C), jnp.int32),
        pltpu.VMEM((CH_L, C), jnp.int32),
        pltpu.VMEM((C, D), jnp.float32),
        pltpu.VMEM((C, D), jnp.float32),
        pltpu.VMEM((C // 8, D), jnp.float32),
        pltpu.SemaphoreType.DMA,
    ],
)


def _tc_dense_body(relu, agg_ref, cnt_ref, x_ref, wl_ref, wr_ref, b_ref, out_ref):
  agg = agg_ref[0] + agg_ref[1]
  cnt = cnt_ref[0, :, 0:1] + cnt_ref[1, :, 0:1]
  mean = agg / jnp.maximum(cnt, 1.0)
  h = (jnp.dot(mean, wl_ref[:], preferred_element_type=jnp.float32)
       + jnp.dot(x_ref[:], wr_ref[:], preferred_element_type=jnp.float32)
       + b_ref[:])
  if relu:
    h = jnp.maximum(h, 0.0)
  out_ref[:] = h


def _tc_dense(agg, cnt, x, w_l, w_r, b, relu):
  grid = 4
  r = N_PAD // grid
  return pl.pallas_call(
      functools.partial(_tc_dense_body, relu),
      grid=(grid,),
      in_specs=[
          pl.BlockSpec((NC, r, D), lambda i: (0, i, 0)),
          pl.BlockSpec((NC, r, D), lambda i: (0, i, 0)),
          pl.BlockSpec((r, D), lambda i: (i, 0)),
          pl.BlockSpec((D, D), lambda i: (0, 0)),
          pl.BlockSpec((D, D), lambda i: (0, 0)),
          pl.BlockSpec((1, D), lambda i: (0, 0)),
      ],
      out_specs=pl.BlockSpec((r, D), lambda i: (i, 0)),
      out_shape=jax.ShapeDtypeStruct((N_PAD, D), jnp.float32),
  )(agg, cnt, x, w_l, w_r, b)


def _tc_reduce_body(part_ref, out_ref):
  cols = [jnp.sum(part_ref[:, q * L:(q + 1) * L], axis=1, keepdims=True)
          for q in range(8)]
  out_ref[:] = jnp.concatenate(cols, axis=1)


def _tc_reduce(part):
  grid = 8
  r = NP_ROWS // grid
  return pl.pallas_call(
      _tc_reduce_body,
      grid=(grid,),
      in_specs=[pl.BlockSpec((r, D), lambda i: (i, 0))],
      out_specs=pl.BlockSpec((r, 8), lambda i: (i, 0)),
      out_shape=jax.ShapeDtypeStruct((NP_ROWS, 8), jnp.float32),
  )(part)


def kernel(node_feature, edge_index, edge_label_index, W1_l, W1_r, b1, W2_l, W2_r, b2):
  ei = edge_index.astype(jnp.int32)
  src = jnp.concatenate(
      [ei[0], jnp.zeros((E_PAD - N_EDGES,), jnp.int32)])
  dst = jnp.concatenate(
      [ei[1], jnp.full((E_PAD - N_EDGES,), TRASH_ROW, jnp.int32)]
  ).reshape(NW, CH_E, C)
  eli = edge_label_index.astype(jnp.int32)
  ia = jnp.concatenate(
      [eli[0], jnp.zeros((NL_PAD - N_LABEL,), jnp.int32)]).reshape(NW, CH_L, C)
  ib = jnp.concatenate(
      [eli[1], jnp.zeros((NL_PAD - N_LABEL,), jnp.int32)]).reshape(NW, CH_L, C)

  x = jnp.concatenate(
      [node_feature, jnp.zeros((N_PAD - N_NODES, D), jnp.float32)])

  cnt = _sc_cnt(dst)
  agg1 = _sc_agg(x, src, dst)
  h1 = _tc_dense(agg1, cnt, x, W1_l, W1_r, b1.reshape(1, D), relu=True)
  agg2 = _sc_agg(h1, src, dst)
  h2 = _tc_dense(agg2, cnt, h1, W2_l, W2_r, b2.reshape(1, D), relu=False)
  part = _sc_score(h2, ia, ib)
  pred = _tc_reduce(part)
  return pred.reshape(NL_PAD)[:N_LABEL]
